# baseline (speedup 1.0000x reference)
import functools

import jax
import jax.numpy as jnp
from jax import lax
from jax.experimental import pallas as pl
from jax.experimental.pallas import tpu as pltpu
from jax.experimental.pallas import tpu_sc as plsc

NODE = 5000
TWO_NODE = 2 * NODE
D = 128
E = 320000
NC, NS = 2, 16
NW = NC * NS
CHUNK = 64
IDX_ROWS = E // CHUNK
NCHUNK = 158
E_PAD = NW * NCHUNK * CHUNK
EPW = E_PAD // NW
E_ALLOC = E_PAD + 2 * CHUNK
ACC_PAD = 10112
ACC_PW = ACC_PAD // NS
DUMP_ROW = TWO_NODE


def _tables_body(x_ref, pw_ref, pb_ref, ws_ref, wd_ref, bd_ref,
                 src_ref, dst_ref):
    h = jnp.dot(x_ref[...], pw_ref[0].T, preferred_element_type=jnp.float32)
    h = h + pb_ref[0, 0]
    src_ref[...] = jnp.dot(h, ws_ref[0].T, preferred_element_type=jnp.float32)
    dst_ref[...] = (jnp.dot(h, wd_ref[0].T, preferred_element_type=jnp.float32)
                    + bd_ref[0, 0])


def _build_tables(x, pw, pb, ws, wd, bd):
    return pl.pallas_call(
        _tables_body,
        grid=(4,),
        in_specs=[
            pl.BlockSpec((NODE, D), lambda i: (i % 2, 0)),
            pl.BlockSpec((1, D, D), lambda i: (i % 2, 0, 0)),
            pl.BlockSpec((1, 1, D), lambda i: (i % 2, 0, 0)),
            pl.BlockSpec((1, D, D), lambda i: (i, 0, 0)),
            pl.BlockSpec((1, D, D), lambda i: (i, 0, 0)),
            pl.BlockSpec((1, 1, D), lambda i: (i, 0, 0)),
        ],
        out_specs=[
            pl.BlockSpec((NODE, D), lambda i: (i, 0)),
            pl.BlockSpec((NODE, D), lambda i: (i, 0)),
        ],
        out_shape=[
            jax.ShapeDtypeStruct((4 * NODE, D), jnp.float32),
            jax.ShapeDtypeStruct((4 * NODE, D), jnp.float32),
        ],
    )(x, pw, pb, ws, wd, bd)


def _idx_body(s_ref, d_ref, si_ref, di_ref):
    s = s_ref[...]
    d = d_ref[...]
    si_ref[...] = s + jnp.where(d >= NODE, TWO_NODE, 0)
    di_ref[...] = d + jnp.where(s >= NODE, TWO_NODE, 0)


def _build_indices(src2d, dst2d):
    return pl.pallas_call(
        _idx_body,
        out_shape=[
            jax.ShapeDtypeStruct(src2d.shape, jnp.int32),
            jax.ShapeDtypeStruct(src2d.shape, jnp.int32),
        ],
    )(src2d, dst2d)


def _edge_body(sidx_hbm, didx_hbm, oidx_hbm, srct_hbm, dstt_hbm, zeros_hbm,
               out_hbm,
               sidx0, sidx1, didx0, didx1, oidx0, oidx1,
               srows0, srows1, drows0, drows1, acc,
               semi0, semi1, semg0, semg1):
    c = lax.axis_index("c")
    s = lax.axis_index("s")
    w = c * NS + s

    sidx = (sidx0, sidx1)
    didx = (didx0, didx1)
    oidx = (oidx0, oidx1)
    srows = (srows0, srows1)
    drows = (drows0, drows1)
    semi = (semi0, semi1)
    semg = (semg0, semg1)

    a0 = pl.multiple_of(s * ACC_PW, 8)
    pltpu.sync_copy(zeros_hbm, acc.at[pl.ds(a0, ACC_PW)])
    plsc.subcore_barrier()

    e0 = pl.multiple_of(w * EPW, 8)

    def idx_descs(i, b):
        base = pl.multiple_of(e0 + i * CHUNK, 8)
        return (
            pltpu.make_async_copy(sidx_hbm.at[pl.ds(base, CHUNK)], sidx[b],
                                  semi[b]),
            pltpu.make_async_copy(didx_hbm.at[pl.ds(base, CHUNK)], didx[b],
                                  semi[b]),
            pltpu.make_async_copy(oidx_hbm.at[pl.ds(base, CHUNK)], oidx[b],
                                  semi[b]),
        )

    def gather_descs(b):
        return (
            pltpu.make_async_copy(srct_hbm.at[sidx[b]], srows[b], semg[b]),
            pltpu.make_async_copy(dstt_hbm.at[didx[b]], drows[b], semg[b]),
        )

    for dsc in idx_descs(0, 0):
        dsc.start()
    for dsc in idx_descs(0, 0):
        dsc.wait()
    for dsc in gather_descs(0):
        dsc.start()
    for dsc in idx_descs(1, 1):
        dsc.start()

    @pl.loop(0, NCHUNK // 2)
    def _pair(g):
        for b in (0, 1):
            i = g * 2 + b
            q = 1 - b
            for dsc in idx_descs(i + 1, q):
                dsc.wait()
            for dsc in gather_descs(q):
                dsc.start()
            for dsc in gather_descs(b):
                dsc.wait()

            @pl.loop(0, CHUNK, unroll=4)
            def _row(r):
                for j in range(D // 16):
                    sl = pl.ds(j * 16, 16)
                    v = srows[b][r, sl] + drows[b][r, sl]
                    e = jnp.exp(v)
                    srows[b][r, sl] = 1.0 - 2.0 / (e + 1.0)

            pltpu.sync_copy(srows[b], acc.at[oidx[b]], add=True)
            for dsc in idx_descs(i + 2, b):
                dsc.start()

    for dsc in gather_descs(0):
        dsc.wait()
    for dsc in idx_descs(NCHUNK + 1, 1):
        dsc.wait()

    plsc.subcore_barrier()
    pltpu.sync_copy(acc.at[pl.ds(a0, ACC_PW)],
                    out_hbm.at[c, pl.ds(a0, ACC_PW)])


@functools.cache
def _edge_stage_fn():
    return functools.partial(
        pl.kernel,
        out_type=jax.ShapeDtypeStruct((NC, ACC_PAD, D), jnp.float32),
        mesh=plsc.VectorSubcoreMesh(core_axis_name="c", subcore_axis_name="s",
                                    num_cores=NC, num_subcores=NS),
        scratch_types=(
            [pltpu.VMEM((CHUNK,), jnp.int32)] * 6
            + [pltpu.VMEM((CHUNK, D), jnp.float32)] * 4
            + [pltpu.VMEM_SHARED((ACC_PAD, D), jnp.float32)]
            + [pltpu.SemaphoreType.DMA] * 4
        ),
    )(_edge_body)


def _inter_body(p_ref, w_ref, b_ref, out_ref):
    m = 0.5 * (p_ref[0] + p_ref[1])
    out_ref[...] = (jnp.dot(m, w_ref[0].T, preferred_element_type=jnp.float32)
                    + b_ref[0, 0])


def _inter(partials, iw, ib):
    return pl.pallas_call(
        _inter_body,
        grid=(2,),
        in_specs=[
            pl.BlockSpec((NC, NODE, D), lambda i: (0, i, 0)),
            pl.BlockSpec((1, D, D), lambda i: (i, 0, 0)),
            pl.BlockSpec((1, 1, D), lambda i: (i, 0, 0)),
        ],
        out_specs=pl.BlockSpec((NODE, D), lambda i: (i, 0)),
        out_shape=jax.ShapeDtypeStruct((TWO_NODE, D), jnp.float32),
    )(partials, iw, ib)


def _one_batch(xb, ei, pw, pb, ws, wd, bd, iw, ib, zeros):
    src_t, dst_t = _build_tables(xb, pw, pb, ws, wd, bd)
    src2d = ei[0].reshape(IDX_ROWS, CHUNK)
    dst2d = ei[1].reshape(IDX_ROWS, CHUNK)
    sidx, didx = _build_indices(src2d, dst2d)
    npad = E_ALLOC - E
    zpad = jnp.zeros((npad,), jnp.int32)
    sidx = jnp.concatenate([sidx.reshape(-1), zpad])
    didx = jnp.concatenate([didx.reshape(-1), zpad])
    oidx = jnp.concatenate([dst2d.reshape(-1),
                            jnp.full((npad,), DUMP_ROW, jnp.int32)])
    partials = _edge_stage_fn()(sidx, didx, oidx, src_t, dst_t, zeros)
    return _inter(partials, iw, ib)


def kernel(x, edge_index, edge_attr, params):
    p = params
    pw = jnp.stack([p['proj_s_W'], p['proj_t_W']])
    pb = jnp.stack([p['proj_s_b'], p['proj_t_b']])[:, None, :]
    ws = jnp.stack([p['intra_s2s_W'], p['intra_t2s_W'],
                    p['intra_s2t_W'], p['intra_t2t_W']])
    wd = jnp.stack([p['intra_s2s_W'], p['intra_s2t_W'],
                    p['intra_t2s_W'], p['intra_t2t_W']])
    bd = 2.0 * jnp.stack([p['intra_s2s_b'], p['intra_s2t_b'],
                          p['intra_t2s_b'], p['intra_t2t_b']])[:, None, :]
    iw = jnp.stack([p['inter_s_W'], p['inter_t_W']])
    ib = jnp.stack([p['inter_s_b'], p['inter_t_b']])[:, None, :]
    zeros = jnp.zeros((ACC_PW, D), jnp.float32)
    outs = [_one_batch(x[bi], edge_index[bi], pw, pb, ws, wd, bd, iw, ib,
                       zeros)
            for bi in range(x.shape[0])]
    return jnp.stack(outs)


# speedup vs baseline: 3.0287x; 3.0287x over previous
import functools

import jax
import jax.numpy as jnp
from jax import lax
from jax.experimental import pallas as pl
from jax.experimental.pallas import tpu as pltpu
from jax.experimental.pallas import tpu_sc as plsc

NODE = 5000
TWO_NODE = 2 * NODE
D = 128
E = 320000
NC, NS = 2, 16
NW = NC * NS
CHUNK = 64
IDX_ROWS = E // D
IDX_ROWS2 = E // D
NCHUNK = 160
E_PAD = NW * NCHUNK * CHUNK
EPW = E_PAD // NW
E_ALLOC = E_PAD + 2 * CHUNK
IDX_ALL = E_ALLOC // D
ACC_PAD = 10112
ACC_PW = ACC_PAD // NS
DUMP_ROW = TWO_NODE


def _tables_body(x_ref, pw_ref, pb_ref, ws_ref, wd_ref, bd_ref,
                 src_ref, dst_ref):
    h = jnp.dot(x_ref[...], pw_ref[0].T, preferred_element_type=jnp.float32)
    h = h + pb_ref[0, 0]
    src_ref[...] = jnp.dot(h, ws_ref[0].T, preferred_element_type=jnp.float32)
    dst_ref[...] = (jnp.dot(h, wd_ref[0].T, preferred_element_type=jnp.float32)
                    + bd_ref[0, 0])


def _build_tables(x, pw, pb, ws, wd, bd):
    return pl.pallas_call(
        _tables_body,
        grid=(4,),
        in_specs=[
            pl.BlockSpec((NODE, D), lambda i: (i % 2, 0)),
            pl.BlockSpec((1, D, D), lambda i: (i % 2, 0, 0)),
            pl.BlockSpec((1, 1, D), lambda i: (i % 2, 0, 0)),
            pl.BlockSpec((1, D, D), lambda i: (i, 0, 0)),
            pl.BlockSpec((1, D, D), lambda i: (i, 0, 0)),
            pl.BlockSpec((1, 1, D), lambda i: (i, 0, 0)),
        ],
        out_specs=[
            pl.BlockSpec((NODE, D), lambda i: (i, 0)),
            pl.BlockSpec((NODE, D), lambda i: (i, 0)),
        ],
        out_shape=[
            jax.ShapeDtypeStruct((4 * NODE, D), jnp.float32),
            jax.ShapeDtypeStruct((4 * NODE, D), jnp.float32),
        ],
    )(x, pw, pb, ws, wd, bd)


def _idx_body(s_ref, d_ref, si_ref, di_ref, oi_ref):
    s = s_ref[...]
    d = d_ref[...]
    npad = IDX_ALL - IDX_ROWS
    zpad = jnp.zeros((npad, D), jnp.int32)
    si_ref[...] = jnp.concatenate(
        [s + jnp.where(d >= NODE, TWO_NODE, 0), zpad])
    di_ref[...] = jnp.concatenate(
        [d + jnp.where(s >= NODE, TWO_NODE, 0), zpad])
    oi_ref[...] = jnp.concatenate(
        [d, jnp.full((npad, D), DUMP_ROW, jnp.int32)])


def _build_indices(src2d, dst2d):
    return pl.pallas_call(
        _idx_body,
        out_shape=[
            jax.ShapeDtypeStruct((IDX_ALL, D), jnp.int32),
            jax.ShapeDtypeStruct((IDX_ALL, D), jnp.int32),
            jax.ShapeDtypeStruct((IDX_ALL, D), jnp.int32),
        ],
    )(src2d, dst2d)


def _edge_body(sidx_hbm, didx_hbm, oidx_hbm, srct_hbm, dstt_hbm, zeros_hbm,
               out_hbm,
               sidx0, sidx1, didx0, didx1, oidx0, oidx1, oidx2, oidx3,
               srows0, srows1, drows0, drows1, acc,
               semi0, semi1, semg0, semg1, semc0, semc1):
    c = lax.axis_index("c")
    s = lax.axis_index("s")
    w = c * NS + s

    sidx = (sidx0, sidx1)
    didx = (didx0, didx1)
    oidx = (oidx0, oidx1, oidx2, oidx3)
    srows = (srows0, srows1)
    drows = (drows0, drows1)
    semi = (semi0, semi1)
    semg = (semg0, semg1)
    semc = (semc0, semc1)

    a0 = pl.multiple_of(s * ACC_PW, 8)
    pltpu.sync_copy(zeros_hbm, acc.at[pl.ds(a0, ACC_PW)])
    plsc.subcore_barrier()

    e0 = pl.multiple_of(w * EPW, 8)

    def idx_descs(i, p, o):
        base = pl.multiple_of(e0 + i * CHUNK, 8)
        return (
            pltpu.make_async_copy(sidx_hbm.at[pl.ds(base, CHUNK)], sidx[p],
                                  semi[p]),
            pltpu.make_async_copy(didx_hbm.at[pl.ds(base, CHUNK)], didx[p],
                                  semi[p]),
            pltpu.make_async_copy(oidx_hbm.at[pl.ds(base, CHUNK)], oidx[o],
                                  semi[p]),
        )

    def gather_descs(p):
        return (
            pltpu.make_async_copy(srct_hbm.at[sidx[p]], srows[p], semg[p]),
            pltpu.make_async_copy(dstt_hbm.at[didx[p]], drows[p], semg[p]),
        )

    def scatter_desc(p, o):
        return pltpu.make_async_copy(srows[p], acc.at[oidx[o]], semc[p])

    for dsc in idx_descs(0, 0, 0):
        dsc.start()
    for dsc in idx_descs(0, 0, 0):
        dsc.wait()
    for dsc in gather_descs(0):
        dsc.start()
    for dsc in idx_descs(1, 1, 1):
        dsc.start()

    @pl.loop(0, NCHUNK // 4)
    def _quad(g):
        for b in (0, 1, 2, 3):
            i = g * 4 + b
            p = b & 1
            q = 1 - p
            for dsc in idx_descs(i + 1, q, (b + 1) % 4):
                dsc.wait()
            if b == 0:
                @pl.when(g > 0)
                def _():
                    scatter_desc(q, 3).wait()
            else:
                scatter_desc(q, (b + 3) % 4).wait()
            for dsc in gather_descs(q):
                dsc.start()
            for dsc in gather_descs(p):
                dsc.wait()

            @pl.loop(0, CHUNK)
            def _row(r):
                for j in range(D // 16):
                    sl = pl.ds(j * 16, 16)
                    v = srows[p][r, sl] + drows[p][r, sl]
                    e = jnp.exp(v)
                    srows[p][r, sl] = 1.0 - 2.0 / (e + 1.0)

            scatter_desc(p, b).start(add=True)
            for dsc in idx_descs(i + 2, p, (b + 2) % 4):
                dsc.start()

    for dsc in gather_descs(0):
        dsc.wait()
    scatter_desc(1, 3).wait()
    for dsc in idx_descs(NCHUNK + 1, 1, 1):
        dsc.wait()

    plsc.subcore_barrier()
    pltpu.sync_copy(acc.at[pl.ds(a0, ACC_PW)],
                    out_hbm.at[c, pl.ds(a0, ACC_PW)])


@functools.cache
def _edge_stage_fn():
    return functools.partial(
        pl.kernel,
        out_type=jax.ShapeDtypeStruct((NC, ACC_PAD, D), jnp.float32),
        mesh=plsc.VectorSubcoreMesh(core_axis_name="c", subcore_axis_name="s",
                                    num_cores=NC, num_subcores=NS),
        scratch_types=(
            [pltpu.VMEM((CHUNK,), jnp.int32)] * 8
            + [pltpu.VMEM((CHUNK, D), jnp.float32)] * 4
            + [pltpu.VMEM_SHARED((ACC_PAD, D), jnp.float32)]
            + [pltpu.SemaphoreType.DMA] * 6
        ),
    )(_edge_body)


def _inter_body(p_ref, w_ref, b_ref, out_ref):
    m = 0.5 * (p_ref[0] + p_ref[1])
    out_ref[...] = (jnp.dot(m, w_ref[0].T, preferred_element_type=jnp.float32)
                    + b_ref[0, 0])


def _inter(partials, iw, ib):
    return pl.pallas_call(
        _inter_body,
        grid=(2,),
        in_specs=[
            pl.BlockSpec((NC, NODE, D), lambda i: (0, i, 0)),
            pl.BlockSpec((1, D, D), lambda i: (i, 0, 0)),
            pl.BlockSpec((1, 1, D), lambda i: (i, 0, 0)),
        ],
        out_specs=pl.BlockSpec((NODE, D), lambda i: (i, 0)),
        out_shape=jax.ShapeDtypeStruct((TWO_NODE, D), jnp.float32),
    )(partials, iw, ib)


def _one_batch(xb, ei, pw, pb, ws, wd, bd, iw, ib, zeros):
    src_t, dst_t = _build_tables(xb, pw, pb, ws, wd, bd)
    src2d = ei[0].reshape(IDX_ROWS2, D)
    dst2d = ei[1].reshape(IDX_ROWS2, D)
    sidx, didx, oidx = _build_indices(src2d, dst2d)
    partials = _edge_stage_fn()(sidx.reshape(-1), didx.reshape(-1),
                                oidx.reshape(-1), src_t, dst_t, zeros)
    return _inter(partials, iw, ib)


def kernel(x, edge_index, edge_attr, params):
    p = params
    pw = jnp.stack([p['proj_s_W'], p['proj_t_W']])
    pb = jnp.stack([p['proj_s_b'], p['proj_t_b']])[:, None, :]
    ws = jnp.stack([p['intra_s2s_W'], p['intra_t2s_W'],
                    p['intra_s2t_W'], p['intra_t2t_W']])
    wd = jnp.stack([p['intra_s2s_W'], p['intra_s2t_W'],
                    p['intra_t2s_W'], p['intra_t2t_W']])
    bd = 2.0 * jnp.stack([p['intra_s2s_b'], p['intra_s2t_b'],
                          p['intra_t2s_b'], p['intra_t2t_b']])[:, None, :]
    iw = jnp.stack([p['inter_s_W'], p['inter_t_W']])
    ib = jnp.stack([p['inter_s_b'], p['inter_t_b']])[:, None, :]
    zeros = jnp.zeros((ACC_PW, D), jnp.float32)
    outs = [_one_batch(x[bi], edge_index[bi], pw, pb, ws, wd, bd, iw, ib,
                       zeros)
            for bi in range(x.shape[0])]
    return jnp.stack(outs)


# speedup vs baseline: 4.9449x; 1.6327x over previous
import functools

import jax
import jax.numpy as jnp
from jax import lax
from jax.experimental import pallas as pl
from jax.experimental.pallas import tpu as pltpu
from jax.experimental.pallas import tpu_sc as plsc

NODE = 5000
TWO_NODE = 2 * NODE
D = 128
E = 320000
NC, NS = 2, 16
NW = NC * NS
CHUNK = 64
IDX_ROWS = E // D
IDX_ROWS2 = E // D
NCHUNK = 160
E_PAD = NW * NCHUNK * CHUNK
EPW = E_PAD // NW
E_ALLOC = E_PAD + 2 * CHUNK
IDX_ALL = E_ALLOC // D
ACC_PAD = 10112
ACC_PW = ACC_PAD // NS
DUMP_ROW = TWO_NODE


def _tables_body(x_ref, pw_ref, pb_ref, ws_ref, wd_ref, bd_ref,
                 src_ref, dst_ref):
    h = jnp.dot(x_ref[...], pw_ref[0].T, preferred_element_type=jnp.float32)
    h = h + pb_ref[0, 0]
    src_ref[...] = jnp.dot(h, ws_ref[0].T, preferred_element_type=jnp.float32)
    dst_ref[...] = (jnp.dot(h, wd_ref[0].T, preferred_element_type=jnp.float32)
                    + bd_ref[0, 0])


def _build_tables(x, pw, pb, ws, wd, bd):
    return pl.pallas_call(
        _tables_body,
        grid=(4,),
        in_specs=[
            pl.BlockSpec((NODE, D), lambda i: (i % 2, 0)),
            pl.BlockSpec((1, D, D), lambda i: (i % 2, 0, 0)),
            pl.BlockSpec((1, 1, D), lambda i: (i % 2, 0, 0)),
            pl.BlockSpec((1, D, D), lambda i: (i, 0, 0)),
            pl.BlockSpec((1, D, D), lambda i: (i, 0, 0)),
            pl.BlockSpec((1, 1, D), lambda i: (i, 0, 0)),
        ],
        out_specs=[
            pl.BlockSpec((NODE, D), lambda i: (i, 0)),
            pl.BlockSpec((NODE, D), lambda i: (i, 0)),
        ],
        out_shape=[
            jax.ShapeDtypeStruct((4 * NODE, D), jnp.float32),
            jax.ShapeDtypeStruct((4 * NODE, D), jnp.float32),
        ],
    )(x, pw, pb, ws, wd, bd)


def _idx_body(s_ref, d_ref, si_ref, di_ref, oi_ref):
    s = s_ref[...]
    d = d_ref[...]
    npad = IDX_ALL - IDX_ROWS
    ii = jax.lax.broadcasted_iota(jnp.int32, (npad, D), 1)
    gpad = ii + jax.lax.broadcasted_iota(jnp.int32, (npad, D), 0) * D
    si_ref[...] = jnp.concatenate(
        [s + jnp.where(d >= NODE, TWO_NODE, 0), gpad % (4 * NODE)])
    di_ref[...] = jnp.concatenate(
        [d + jnp.where(s >= NODE, TWO_NODE, 0), gpad % (4 * NODE)])
    oi_ref[...] = jnp.concatenate(
        [d, DUMP_ROW + gpad % (ACC_PAD - DUMP_ROW)])


def _build_indices(src2d, dst2d):
    return pl.pallas_call(
        _idx_body,
        out_shape=[
            jax.ShapeDtypeStruct((IDX_ALL, D), jnp.int32),
            jax.ShapeDtypeStruct((IDX_ALL, D), jnp.int32),
            jax.ShapeDtypeStruct((IDX_ALL, D), jnp.int32),
        ],
    )(src2d, dst2d)


def _edge_body(sidx_hbm, didx_hbm, oidx_hbm, srct_hbm, dstt_hbm, zeros_hbm,
               out_hbm,
               sidx0, sidx1, didx0, didx1, oidx0, oidx1, oidx2, oidx3,
               srows0, srows1, drows0, drows1, acc,
               semi0, semi1, semg0, semg1, semc0, semc1):
    c = lax.axis_index("c")
    s = lax.axis_index("s")
    w = c * NS + s

    sidx = (sidx0, sidx1)
    didx = (didx0, didx1)
    oidx = (oidx0, oidx1, oidx2, oidx3)
    srows = (srows0, srows1)
    drows = (drows0, drows1)
    semi = (semi0, semi1)
    semg = (semg0, semg1)
    semc = (semc0, semc1)

    a0 = pl.multiple_of(s * ACC_PW, 8)
    pltpu.sync_copy(zeros_hbm, acc.at[pl.ds(a0, ACC_PW)])
    plsc.subcore_barrier()

    e0 = pl.multiple_of(w * EPW, 8)

    def idx_descs(i, p, o):
        base = pl.multiple_of(e0 + i * CHUNK, 8)
        return (
            pltpu.make_async_copy(sidx_hbm.at[pl.ds(base, CHUNK)], sidx[p],
                                  semi[p]),
            pltpu.make_async_copy(didx_hbm.at[pl.ds(base, CHUNK)], didx[p],
                                  semi[p]),
            pltpu.make_async_copy(oidx_hbm.at[pl.ds(base, CHUNK)], oidx[o],
                                  semi[p]),
        )

    def gather_descs(p):
        return (
            pltpu.make_async_copy(srct_hbm.at[sidx[p]], srows[p], semg[p]),
            pltpu.make_async_copy(dstt_hbm.at[didx[p]], drows[p], semg[p]),
        )

    def scatter_desc(p, o):
        return pltpu.make_async_copy(srows[p], acc.at[oidx[o]], semc[p])

    for dsc in idx_descs(0, 0, 0):
        dsc.start()
    for dsc in idx_descs(0, 0, 0):
        dsc.wait()
    for dsc in gather_descs(0):
        dsc.start()
    for dsc in idx_descs(1, 1, 1):
        dsc.start()

    @pl.loop(0, NCHUNK // 4)
    def _quad(g):
        for b in (0, 1, 2, 3):
            i = g * 4 + b
            p = b & 1
            q = 1 - p
            for dsc in idx_descs(i + 1, q, (b + 1) % 4):
                dsc.wait()
            if b == 0:
                @pl.when(g > 0)
                def _():
                    scatter_desc(q, 3).wait()
            else:
                scatter_desc(q, (b + 3) % 4).wait()
            for dsc in gather_descs(q):
                dsc.start()
            for dsc in gather_descs(p):
                dsc.wait()

            @pl.loop(0, CHUNK)
            def _row(r):
                for j in range(D // 16):
                    sl = pl.ds(j * 16, 16)
                    v = srows[p][r, sl] + drows[p][r, sl]
                    e = jnp.exp(v)
                    srows[p][r, sl] = 1.0 - 2.0 / (e + 1.0)

            scatter_desc(p, b).start(add=True)
            for dsc in idx_descs(i + 2, p, (b + 2) % 4):
                dsc.start()

    for dsc in gather_descs(0):
        dsc.wait()
    scatter_desc(1, 3).wait()
    for dsc in idx_descs(NCHUNK + 1, 1, 1):
        dsc.wait()

    plsc.subcore_barrier()
    pltpu.sync_copy(acc.at[pl.ds(a0, ACC_PW)],
                    out_hbm.at[c, pl.ds(a0, ACC_PW)])


@functools.cache
def _edge_stage_fn():
    return functools.partial(
        pl.kernel,
        out_type=jax.ShapeDtypeStruct((NC, ACC_PAD, D), jnp.float32),
        mesh=plsc.VectorSubcoreMesh(core_axis_name="c", subcore_axis_name="s",
                                    num_cores=NC, num_subcores=NS),
        scratch_types=(
            [pltpu.VMEM((CHUNK,), jnp.int32)] * 8
            + [pltpu.VMEM((CHUNK, D), jnp.float32)] * 4
            + [pltpu.VMEM_SHARED((ACC_PAD, D), jnp.float32)]
            + [pltpu.SemaphoreType.DMA] * 6
        ),
    )(_edge_body)


def _inter_body(p_ref, w_ref, b_ref, out_ref):
    m = 0.5 * (p_ref[0] + p_ref[1])
    out_ref[...] = (jnp.dot(m, w_ref[0].T, preferred_element_type=jnp.float32)
                    + b_ref[0, 0])


def _inter(partials, iw, ib):
    return pl.pallas_call(
        _inter_body,
        grid=(2,),
        in_specs=[
            pl.BlockSpec((NC, NODE, D), lambda i: (0, i, 0)),
            pl.BlockSpec((1, D, D), lambda i: (i, 0, 0)),
            pl.BlockSpec((1, 1, D), lambda i: (i, 0, 0)),
        ],
        out_specs=pl.BlockSpec((NODE, D), lambda i: (i, 0)),
        out_shape=jax.ShapeDtypeStruct((TWO_NODE, D), jnp.float32),
    )(partials, iw, ib)


def _one_batch(xb, ei, pw, pb, ws, wd, bd, iw, ib, zeros):
    src_t, dst_t = _build_tables(xb, pw, pb, ws, wd, bd)
    src2d = ei[0].reshape(IDX_ROWS2, D)
    dst2d = ei[1].reshape(IDX_ROWS2, D)
    sidx, didx, oidx = _build_indices(src2d, dst2d)
    partials = _edge_stage_fn()(sidx.reshape(-1), didx.reshape(-1),
                                oidx.reshape(-1), src_t, dst_t, zeros)
    return _inter(partials, iw, ib)


def kernel(x, edge_index, edge_attr, params):
    p = params
    pw = jnp.stack([p['proj_s_W'], p['proj_t_W']])
    pb = jnp.stack([p['proj_s_b'], p['proj_t_b']])[:, None, :]
    ws = jnp.stack([p['intra_s2s_W'], p['intra_t2s_W'],
                    p['intra_s2t_W'], p['intra_t2t_W']])
    wd = jnp.stack([p['intra_s2s_W'], p['intra_s2t_W'],
                    p['intra_t2s_W'], p['intra_t2t_W']])
    bd = 2.0 * jnp.stack([p['intra_s2s_b'], p['intra_s2t_b'],
                          p['intra_t2s_b'], p['intra_t2t_b']])[:, None, :]
    iw = jnp.stack([p['inter_s_W'], p['inter_t_W']])
    ib = jnp.stack([p['inter_s_b'], p['inter_t_b']])[:, None, :]
    zeros = jnp.zeros((ACC_PW, D), jnp.float32)
    outs = [_one_batch(x[bi], edge_index[bi], pw, pb, ws, wd, bd, iw, ib,
                       zeros)
            for bi in range(x.shape[0])]
    return jnp.stack(outs)


# speedup vs baseline: 5.1911x; 1.0498x over previous
import functools

import jax
import jax.numpy as jnp
from jax import lax
from jax.experimental import pallas as pl
from jax.experimental.pallas import tpu as pltpu
from jax.experimental.pallas import tpu_sc as plsc

NODE = 5000
TWO_NODE = 2 * NODE
D = 128
E = 320000
NC, NS = 2, 16
NW = NC * NS
CHUNK = 96
IDX_ROWS = E // D
IDX_ROWS2 = E // D
NCHUNK = 108
E_PAD = NW * NCHUNK * CHUNK
EPW = E_PAD // NW
E_ALLOC = E_PAD + 2 * CHUNK
IDX_ALL = -(-E_ALLOC // D)
ACC_PAD = 10112
ACC_PW = ACC_PAD // NS
DUMP_ROW = TWO_NODE


def _tables_body(x_ref, pw_ref, pb_ref, ws_ref, wd_ref, bd_ref,
                 src_ref, dst_ref):
    h = jnp.dot(x_ref[...], pw_ref[0].T, preferred_element_type=jnp.float32)
    h = h + pb_ref[0, 0]
    src_ref[...] = jnp.dot(h, ws_ref[0].T, preferred_element_type=jnp.float32)
    dst_ref[...] = (jnp.dot(h, wd_ref[0].T, preferred_element_type=jnp.float32)
                    + bd_ref[0, 0])


def _build_tables(x, pw, pb, ws, wd, bd):
    return pl.pallas_call(
        _tables_body,
        grid=(4,),
        in_specs=[
            pl.BlockSpec((NODE, D), lambda i: (i % 2, 0)),
            pl.BlockSpec((1, D, D), lambda i: (i % 2, 0, 0)),
            pl.BlockSpec((1, 1, D), lambda i: (i % 2, 0, 0)),
            pl.BlockSpec((1, D, D), lambda i: (i, 0, 0)),
            pl.BlockSpec((1, D, D), lambda i: (i, 0, 0)),
            pl.BlockSpec((1, 1, D), lambda i: (i, 0, 0)),
        ],
        out_specs=[
            pl.BlockSpec((NODE, D), lambda i: (i, 0)),
            pl.BlockSpec((NODE, D), lambda i: (i, 0)),
        ],
        out_shape=[
            jax.ShapeDtypeStruct((4 * NODE, D), jnp.float32),
            jax.ShapeDtypeStruct((4 * NODE, D), jnp.float32),
        ],
    )(x, pw, pb, ws, wd, bd)


def _idx_body(s_ref, d_ref, si_ref, di_ref, oi_ref):
    s = s_ref[...]
    d = d_ref[...]
    npad = IDX_ALL - IDX_ROWS
    ii = jax.lax.broadcasted_iota(jnp.int32, (npad, D), 1)
    gpad = ii + jax.lax.broadcasted_iota(jnp.int32, (npad, D), 0) * D
    si_ref[...] = jnp.concatenate(
        [s + jnp.where(d >= NODE, TWO_NODE, 0), gpad % (4 * NODE)])
    di_ref[...] = jnp.concatenate(
        [d + jnp.where(s >= NODE, TWO_NODE, 0), gpad % (4 * NODE)])
    oi_ref[...] = jnp.concatenate(
        [d, DUMP_ROW + gpad % (ACC_PAD - DUMP_ROW)])


def _build_indices(src2d, dst2d):
    return pl.pallas_call(
        _idx_body,
        out_shape=[
            jax.ShapeDtypeStruct((IDX_ALL, D), jnp.int32),
            jax.ShapeDtypeStruct((IDX_ALL, D), jnp.int32),
            jax.ShapeDtypeStruct((IDX_ALL, D), jnp.int32),
        ],
    )(src2d, dst2d)


def _edge_body(sidx_hbm, didx_hbm, oidx_hbm, srct_hbm, dstt_hbm, zeros_hbm,
               out_hbm,
               sidx0, sidx1, didx0, didx1, oidx0, oidx1, oidx2, oidx3,
               srows0, srows1, drows0, drows1, acc,
               semi0, semi1, semg0, semg1, semc0, semc1):
    c = lax.axis_index("c")
    s = lax.axis_index("s")
    w = c * NS + s

    sidx = (sidx0, sidx1)
    didx = (didx0, didx1)
    oidx = (oidx0, oidx1, oidx2, oidx3)
    srows = (srows0, srows1)
    drows = (drows0, drows1)
    semi = (semi0, semi1)
    semg = (semg0, semg1)
    semc = (semc0, semc1)

    a0 = pl.multiple_of(s * ACC_PW, 8)
    pltpu.sync_copy(zeros_hbm, acc.at[pl.ds(a0, ACC_PW)])
    plsc.subcore_barrier()

    e0 = pl.multiple_of(w * EPW, 8)

    def idx_descs(i, p, o):
        base = pl.multiple_of(e0 + i * CHUNK, 8)
        return (
            pltpu.make_async_copy(sidx_hbm.at[pl.ds(base, CHUNK)], sidx[p],
                                  semi[p]),
            pltpu.make_async_copy(didx_hbm.at[pl.ds(base, CHUNK)], didx[p],
                                  semi[p]),
            pltpu.make_async_copy(oidx_hbm.at[pl.ds(base, CHUNK)], oidx[o],
                                  semi[p]),
        )

    def gather_descs(p):
        return (
            pltpu.make_async_copy(srct_hbm.at[sidx[p]], srows[p], semg[p]),
            pltpu.make_async_copy(dstt_hbm.at[didx[p]], drows[p], semg[p]),
        )

    def scatter_desc(p, o):
        return pltpu.make_async_copy(srows[p], acc.at[oidx[o]], semc[p])

    for dsc in idx_descs(0, 0, 0):
        dsc.start()
    for dsc in idx_descs(0, 0, 0):
        dsc.wait()
    for dsc in gather_descs(0):
        dsc.start()
    for dsc in idx_descs(1, 1, 1):
        dsc.start()

    @pl.loop(0, NCHUNK // 4)
    def _quad(g):
        for b in (0, 1, 2, 3):
            i = g * 4 + b
            p = b & 1
            q = 1 - p
            for dsc in idx_descs(i + 1, q, (b + 1) % 4):
                dsc.wait()
            if b == 0:
                @pl.when(g > 0)
                def _():
                    scatter_desc(q, 3).wait()
            else:
                scatter_desc(q, (b + 3) % 4).wait()
            for dsc in gather_descs(q):
                dsc.start()
            for dsc in gather_descs(p):
                dsc.wait()

            @pl.loop(0, CHUNK)
            def _row(r):
                for j in range(D // 16):
                    sl = pl.ds(j * 16, 16)
                    v = srows[p][r, sl] + drows[p][r, sl]
                    e = jnp.exp(v)
                    srows[p][r, sl] = 1.0 - 2.0 / (e + 1.0)

            scatter_desc(p, b).start(add=True)
            for dsc in idx_descs(i + 2, p, (b + 2) % 4):
                dsc.start()

    for dsc in gather_descs(0):
        dsc.wait()
    scatter_desc(1, 3).wait()
    for dsc in idx_descs(NCHUNK + 1, 1, 1):
        dsc.wait()

    plsc.subcore_barrier()
    pltpu.sync_copy(acc.at[pl.ds(a0, ACC_PW)],
                    out_hbm.at[c, pl.ds(a0, ACC_PW)])


@functools.cache
def _edge_stage_fn():
    return functools.partial(
        pl.kernel,
        out_type=jax.ShapeDtypeStruct((NC, ACC_PAD, D), jnp.float32),
        mesh=plsc.VectorSubcoreMesh(core_axis_name="c", subcore_axis_name="s",
                                    num_cores=NC, num_subcores=NS),
        scratch_types=(
            [pltpu.VMEM((CHUNK,), jnp.int32)] * 8
            + [pltpu.VMEM((CHUNK, D), jnp.float32)] * 4
            + [pltpu.VMEM_SHARED((ACC_PAD, D), jnp.float32)]
            + [pltpu.SemaphoreType.DMA] * 6
        ),
    )(_edge_body)


def _inter_body(p_ref, w_ref, b_ref, out_ref):
    m = 0.5 * (p_ref[0] + p_ref[1])
    out_ref[...] = (jnp.dot(m, w_ref[0].T, preferred_element_type=jnp.float32)
                    + b_ref[0, 0])


def _inter(partials, iw, ib):
    return pl.pallas_call(
        _inter_body,
        grid=(2,),
        in_specs=[
            pl.BlockSpec((NC, NODE, D), lambda i: (0, i, 0)),
            pl.BlockSpec((1, D, D), lambda i: (i, 0, 0)),
            pl.BlockSpec((1, 1, D), lambda i: (i, 0, 0)),
        ],
        out_specs=pl.BlockSpec((NODE, D), lambda i: (i, 0)),
        out_shape=jax.ShapeDtypeStruct((TWO_NODE, D), jnp.float32),
    )(partials, iw, ib)


def _one_batch(xb, ei, pw, pb, ws, wd, bd, iw, ib, zeros):
    src_t, dst_t = _build_tables(xb, pw, pb, ws, wd, bd)
    src2d = ei[0].reshape(IDX_ROWS2, D)
    dst2d = ei[1].reshape(IDX_ROWS2, D)
    sidx, didx, oidx = _build_indices(src2d, dst2d)
    partials = _edge_stage_fn()(sidx.reshape(-1), didx.reshape(-1),
                                oidx.reshape(-1), src_t, dst_t, zeros)
    return _inter(partials, iw, ib)


def kernel(x, edge_index, edge_attr, params):
    p = params
    pw = jnp.stack([p['proj_s_W'], p['proj_t_W']])
    pb = jnp.stack([p['proj_s_b'], p['proj_t_b']])[:, None, :]
    ws = jnp.stack([p['intra_s2s_W'], p['intra_t2s_W'],
                    p['intra_s2t_W'], p['intra_t2t_W']])
    wd = jnp.stack([p['intra_s2s_W'], p['intra_s2t_W'],
                    p['intra_t2s_W'], p['intra_t2t_W']])
    bd = 2.0 * jnp.stack([p['intra_s2s_b'], p['intra_s2t_b'],
                          p['intra_t2s_b'], p['intra_t2t_b']])[:, None, :]
    iw = jnp.stack([p['inter_s_W'], p['inter_t_W']])
    ib = jnp.stack([p['inter_s_b'], p['inter_t_b']])[:, None, :]
    zeros = jnp.zeros((ACC_PW, D), jnp.float32)
    outs = [_one_batch(x[bi], edge_index[bi], pw, pb, ws, wd, bd, iw, ib,
                       zeros)
            for bi in range(x.shape[0])]
    return jnp.stack(outs)
